# baseline (speedup 1.0000x reference)
import functools

import jax
import jax.numpy as jnp
from jax.experimental import pallas as pl

_B = 4
_C = 384
_HW = 224 * 224
_R = 48
_E = 64
_K = 8
_EPS = 1e-5

_RBLK = 32


def _pool_body(x_ref, o_ref):
    s = jnp.sum(x_ref[...], axis=1)
    o_ref[0, 0, :] = jnp.sum(s, axis=1)


def _route_body(ps_ref, w1_ref, b1_ref, w2_ref, b2_ref, vals_ref, idx_ref):
    pooled = ps_ref[...]
    h = jax.lax.dot_general(pooled, w1_ref[...], (((1,), (1,)), ((), ())),
                            preferred_element_type=jnp.float32)
    h = h + b1_ref[...]
    h = h * jax.nn.sigmoid(h)
    logits = jax.lax.dot_general(h, w2_ref[...], (((1,), (1,)), ((), ())),
                                 preferred_element_type=jnp.float32)
    logits = logits + b2_ref[...]
    m = jnp.max(logits, axis=1, keepdims=True)
    e = jnp.exp(logits - m)
    probs = e / jnp.sum(e, axis=1, keepdims=True)

    iota = jax.lax.broadcasted_iota(jnp.int32, (_B, _E), 1)
    p = probs
    vals = []
    idxs = []
    for _ in range(_K):
        mx = jnp.max(p, axis=1, keepdims=True)
        sel = jnp.min(jnp.where(p == mx, iota, _E), axis=1, keepdims=True)
        vals.append(mx)
        idxs.append(sel)
        p = jnp.where(iota == sel, -jnp.inf, p)
    v = jnp.concatenate(vals, axis=1)
    i = jnp.concatenate(idxs, axis=1)
    s = jnp.sum(v, axis=1, keepdims=True) + 1e-6
    vals_ref[...] = v / s
    idx_ref[...] = i


@jax.jit
def kernel(x, W1, gamma1, beta1, mean1, var1, W2, gamma2, beta2, mean2, var2):
    s1 = gamma1 * jax.lax.rsqrt(var1 + _EPS)
    s2 = gamma2 * jax.lax.rsqrt(var2 + _EPS)
    w1 = (W1 * s1[:, None]) * (1.0 / _HW)
    b1 = (beta1 - mean1 * s1)[None, :]
    w2 = W2 * s2[:, None]
    b2 = (beta2 - mean2 * s2)[None, :]

    rows = _B * _C
    xv = x.reshape(rows, 392, 128)
    n_steps = rows // _RBLK
    pooled_sums = pl.pallas_call(
        _pool_body,
        grid=(n_steps,),
        in_specs=[pl.BlockSpec((_RBLK, 392, 128), lambda i: (i, 0, 0))],
        out_specs=pl.BlockSpec((1, 1, _RBLK), lambda i: (i, 0, 0)),
        out_shape=jax.ShapeDtypeStruct((n_steps, 1, _RBLK), jnp.float32),
    )(xv).reshape(_B, _C)

    vals, idxs = pl.pallas_call(
        _route_body,
        out_shape=(
            jax.ShapeDtypeStruct((_B, _K), jnp.float32),
            jax.ShapeDtypeStruct((_B, _K), jnp.int32),
        ),
    )(pooled_sums, w1, b1, w2, b2)
    return vals, idxs


# speedup vs baseline: 7.9193x; 7.9193x over previous
import functools

import jax
import jax.numpy as jnp
from jax.experimental import pallas as pl

_B = 4
_C = 384
_HW = 224 * 224
_R = 48
_E = 64
_K = 8
_EPS = 1e-5

_HBLK = 28


def _pool_body(x_ref, o_ref):
    b = pl.program_id(0)
    h = pl.program_id(1)
    s = jnp.sum(x_ref[0], axis=0)
    part = jnp.sum(s, axis=0)

    @pl.when(h == 0)
    def _init():
        o_ref[b, :] = part

    @pl.when(h != 0)
    def _acc():
        o_ref[b, :] += part


def _route_body(ps_ref, w1_ref, b1_ref, w2_ref, b2_ref, vals_ref, idx_ref):
    pooled = ps_ref[...]
    h = jax.lax.dot_general(pooled, w1_ref[...], (((1,), (1,)), ((), ())),
                            preferred_element_type=jnp.float32)
    h = h + b1_ref[...]
    h = h * jax.nn.sigmoid(h)
    logits = jax.lax.dot_general(h, w2_ref[...], (((1,), (1,)), ((), ())),
                                 preferred_element_type=jnp.float32)
    logits = logits + b2_ref[...]
    m = jnp.max(logits, axis=1, keepdims=True)
    e = jnp.exp(logits - m)
    probs = e / jnp.sum(e, axis=1, keepdims=True)

    iota = jax.lax.broadcasted_iota(jnp.int32, (_B, _E), 1)
    p = probs
    vals = []
    idxs = []
    for _ in range(_K):
        mx = jnp.max(p, axis=1, keepdims=True)
        sel = jnp.min(jnp.where(p == mx, iota, _E), axis=1, keepdims=True)
        vals.append(mx)
        idxs.append(sel)
        p = jnp.where(iota == sel, -jnp.inf, p)
    v = jnp.concatenate(vals, axis=1)
    i = jnp.concatenate(idxs, axis=1)
    s = jnp.sum(v, axis=1, keepdims=True) + 1e-6
    vals_ref[...] = v / s
    idx_ref[...] = i


@jax.jit
def kernel(x, W1, gamma1, beta1, mean1, var1, W2, gamma2, beta2, mean2, var2):
    s1 = gamma1 * jax.lax.rsqrt(var1 + _EPS)
    s2 = gamma2 * jax.lax.rsqrt(var2 + _EPS)
    w1 = (W1 * s1[:, None]) * (1.0 / _HW)
    b1 = (beta1 - mean1 * s1)[None, :]
    w2 = W2 * s2[:, None]
    b2 = (beta2 - mean2 * s2)[None, :]

    xt = jnp.transpose(x, (0, 2, 3, 1))
    pooled_sums = pl.pallas_call(
        _pool_body,
        grid=(_B, 224 // _HBLK),
        in_specs=[pl.BlockSpec((1, _HBLK, 224, _C), lambda b, h: (b, h, 0, 0))],
        out_specs=pl.BlockSpec((_B, _C), lambda b, h: (0, 0)),
        out_shape=jax.ShapeDtypeStruct((_B, _C), jnp.float32),
    )(xt)

    vals, idxs = pl.pallas_call(
        _route_body,
        out_shape=(
            jax.ShapeDtypeStruct((_B, _K), jnp.float32),
            jax.ShapeDtypeStruct((_B, _K), jnp.int32),
        ),
    )(pooled_sums, w1, b1, w2, b2)
    return vals, idxs
